# Initial kernel scaffold; baseline (speedup 1.0000x reference)
import jax
import jax.numpy as jnp
from jax.experimental import pallas as pl


def kernel(x, edge_index, edge_weight, W0, W1, W2):
    raise NotImplementedError("write your pallas kernel here")


# speedup vs baseline: 3.5754x; 3.5754x over previous
import functools

import jax
import jax.numpy as jnp
from jax import lax
from jax.experimental import pallas as pl
from jax.experimental.pallas import tpu as pltpu
from jax.experimental.pallas import tpu_sc as plsc

N = 10000
E = 320000
D = 128
C = 0.4
C_LIN = 1.0

NC = 2
NS = 16
NW = NC * NS
K = 128
EPW = -(-E // NW)
CH = -(-EPW // K)
E_PAD = NW * CH * K

N_PAD = 10240
ROWS_PER_TILE = N_PAD // NS


def _norm(x):
    return jnp.sqrt(jnp.clip(jnp.sum(x * x, axis=-1, keepdims=True), 1e-15, None))


def _artanh(x):
    x = jnp.clip(x, -1.0 + 1e-7, 1.0 - 1e-7)
    return 0.5 * jnp.log((1.0 + x) / (1.0 - x))


def _expmap0(u, c):
    sc = jnp.sqrt(c)
    n = _norm(u)
    return jnp.tanh(sc * n) * u / (sc * n)


def _logmap0(p, c):
    sc = jnp.sqrt(c)
    n = _norm(p)
    return _artanh(sc * n) * p / (sc * n)


def _proj(x, c):
    maxn = (1.0 - 1e-3) / jnp.sqrt(c)
    n = _norm(x)
    return jnp.where(n > maxn, x / n * maxn, x)


def _pre(h, W):
    ht = _logmap0(h, C)
    u = _logmap0(ht, C_LIN)
    y = jnp.dot(u, W, preferred_element_type=jnp.float32)
    return _proj(_expmap0(y, C_LIN), C_LIN)


def _post(p):
    s = p[0] + p[1]
    h2 = _proj(_expmap0(s, C), C)
    h3 = jax.nn.relu(_logmap0(h2, C))
    return _proj(_expmap0(h3, C), C)


BLK = 2000


def _entry_body(x_ref, w_ref, o_ref):
    h = _expmap0(x_ref[...], C)
    o_ref[...] = _pre(h, w_ref[...])


def _mid_body(p_ref, w_ref, o_ref):
    o_ref[...] = _pre(_post(p_ref[...]), w_ref[...])


def _final_body(p_ref, o_ref):
    o_ref[...] = _post(p_ref[...])


def _entry(x, W):
    return pl.pallas_call(
        _entry_body,
        grid=(N // BLK,),
        in_specs=[
            pl.BlockSpec((BLK, D), lambda i: (i, 0)),
            pl.BlockSpec((D, D), lambda i: (0, 0)),
        ],
        out_specs=pl.BlockSpec((BLK, D), lambda i: (i, 0)),
        out_shape=jax.ShapeDtypeStruct((N, D), jnp.float32),
    )(x, W)


def _mid(p, W):
    return pl.pallas_call(
        _mid_body,
        grid=(N // BLK,),
        in_specs=[
            pl.BlockSpec((NC, BLK, D), lambda i: (0, i, 0)),
            pl.BlockSpec((D, D), lambda i: (0, 0)),
        ],
        out_specs=pl.BlockSpec((BLK, D), lambda i: (i, 0)),
        out_shape=jax.ShapeDtypeStruct((N, D), jnp.float32),
    )(p, W)


def _final(p):
    return pl.pallas_call(
        _final_body,
        grid=(N // BLK,),
        in_specs=[pl.BlockSpec((NC, BLK, D), lambda i: (0, i, 0))],
        out_specs=pl.BlockSpec((BLK, D), lambda i: (i, 0)),
        out_shape=jax.ShapeDtypeStruct((N, D), jnp.float32),
    )(p)


@functools.cache
def _make_spmm():
    mesh = plsc.VectorSubcoreMesh(core_axis_name="c", subcore_axis_name="s")
    return functools.partial(
        pl.kernel,
        mesh=mesh,
        out_type=jax.ShapeDtypeStruct((NC, N_PAD, D), jnp.float32),
        scratch_types=[
            pltpu.VMEM((CH, K), jnp.int32),
            pltpu.VMEM((CH, K), jnp.int32),
            pltpu.VMEM((K * 16,), jnp.float32),
            pltpu.VMEM((K, D), jnp.float32),
            pltpu.VMEM_SHARED((N_PAD, D), jnp.float32),
            pltpu.SemaphoreType.DMA,
        ],
    )(_spmm_body)


def _spmm_body(hl_hbm, src_hbm, dst_hbm, wb_hbm, out_hbm,
               src_v, dst_v, wb_v, rows_v, acc_sh, sem):
    cid = lax.axis_index("c")
    sid = lax.axis_index("s")
    tid = sid * NC + cid

    pltpu.sync_copy(src_hbm.at[tid], src_v)
    pltpu.sync_copy(dst_hbm.at[tid], dst_v)

    def _zbody(i, _):
        for cc in range(D // 16):
            rows_v[i, pl.ds(cc * 16, 16)] = jnp.zeros((16,), jnp.float32)
        return 0
    lax.fori_loop(0, K, _zbody, 0)

    row0 = sid * ROWS_PER_TILE
    for b in range(ROWS_PER_TILE // K):
        pltpu.sync_copy(rows_v, acc_sh.at[pl.ds(row0 + b * K, K)])
    plsc.subcore_barrier()

    def _chunk(j, _):
        gcp = pltpu.async_copy(hl_hbm.at[src_v.at[j]], rows_v, sem)
        base = (tid * CH + j) * (K * 16)
        pltpu.sync_copy(wb_hbm.at[pl.ds(base, K * 16)], wb_v)
        gcp.wait()

        def _scale(i, _):
            wb = wb_v[pl.ds(i * 16, 16)]
            for cc in range(D // 16):
                sl = pl.ds(cc * 16, 16)
                rows_v[i, sl] = rows_v[i, sl] * wb
            return 0
        lax.fori_loop(0, K, _scale, 0)

        pltpu.sync_copy(rows_v, acc_sh.at[dst_v.at[j]], add=True)
        return 0
    lax.fori_loop(0, CH, _chunk, 0)

    plsc.subcore_barrier()
    pltpu.sync_copy(acc_sh.at[pl.ds(row0, ROWS_PER_TILE)],
                    out_hbm.at[cid, pl.ds(row0, ROWS_PER_TILE)])


def kernel(x, edge_index, edge_weight, W0, W1, W2):
    pad = E_PAD - E
    src = jnp.pad(edge_index[0], (0, pad)).reshape(NW, CH, K)
    dst = jnp.pad(edge_index[1], (0, pad)).reshape(NW, CH, K)
    w = jnp.repeat(jnp.pad(edge_weight, (0, pad)), 16)

    spmm = _make_spmm()
    hl = _entry(x, W0)
    p = spmm(hl, src, dst, w)
    hl = _mid(p, W1)
    p = spmm(hl, src, dst, w)
    hl = _mid(p, W2)
    p = spmm(hl, src, dst, w)
    return _final(p)
